# baseline (speedup 1.0000x reference)
import functools

import jax
import jax.numpy as jnp
from jax import lax
from jax.experimental import pallas as pl
from jax.experimental.pallas import tpu as pltpu
from jax.experimental.pallas import tpu_sc as plsc

NC = 2
NS = 16
NW = NC * NS
K = 128

_vector_mesh = plsc.VectorSubcoreMesh(core_axis_name="c", subcore_axis_name="s")


def _deg_body(blk, rpt, dst_hbm, deg_out, idx_v, ones_v, z_v, deg_sh):
    c = lax.axis_index("c")
    s = lax.axis_index("s")
    slab = c * NS + s
    z16 = jnp.zeros((16,), jnp.float32)
    o16 = jnp.ones((16,), jnp.float32)
    for i in range(K // 16):
        ones_v[pl.ds(i * 16, 16)] = o16
    for i in range(rpt // 16):
        z_v[pl.ds(i * 16, 16)] = z16
    pltpu.sync_copy(z_v, deg_sh.at[pl.ds(s * rpt, rpt)])
    pltpu.sync_copy(dst_hbm.at[slab], idx_v)
    plsc.subcore_barrier()

    @pl.loop(0, blk)
    def _(j):
        pltpu.sync_copy(ones_v, deg_sh.at[idx_v.at[j]], add=True)

    plsc.subcore_barrier()
    pltpu.sync_copy(deg_sh.at[pl.ds(s * rpt, rpt)],
                    deg_out.at[c, pl.ds(s * rpt, rpt)])


def _agg_body(blk, rpt, d, hw_hbm, src_hbm, dst_hbm, out_hbm,
              src_v, dsta, dstb, bufa, bufb, z_v, acc_sh,
              sga, sgb, sda, sdb):
    CH = 8
    npair = blk // (2 * CH)
    c = lax.axis_index("c")
    s = lax.axis_index("s")
    slab = c * NS + s
    z16 = jnp.zeros((16,), jnp.float32)
    for r in range(16):
        for i in range(d // 16):
            z_v[r, pl.ds(i * 16, 16)] = z16

    @pl.loop(0, rpt // 16)
    def _(i):
        pltpu.sync_copy(z_v, acc_sh.at[pl.ds(s * rpt + i * 16, 16)])

    pltpu.sync_copy(src_hbm.at[slab], src_v)
    pltpu.sync_copy(dst_hbm.at[slab, pl.ds(0, CH)], dsta)
    plsc.subcore_barrier()

    gbufs = (bufa, bufb)
    gsems = (sga, sgb)
    pltpu.async_copy(hw_hbm.at[src_v.at[0]], bufa, sga)

    @pl.loop(0, npair)
    def _(g):
        for half, (dcur, dnxt, scur, snxt) in enumerate(
                ((dsta, dstb, sda, sdb), (dstb, dsta, sdb, sda))):
            ch = g * 2 + half
            if half == 0:
                pltpu.async_copy(
                    dst_hbm.at[slab, pl.ds((ch + 1) * CH, CH)], dnxt, snxt)
            else:
                @pl.when(g < npair - 1)
                def _():
                    pltpu.async_copy(
                        dst_hbm.at[slab, pl.ds((ch + 1) * CH, CH)], dnxt, snxt)
            if half == 0:
                @pl.when(g > 0)
                def _():
                    pltpu.make_async_copy(
                        dst_hbm.at[slab, pl.ds(ch * CH, CH)], dcur, scur).wait()
            else:
                pltpu.make_async_copy(
                    dst_hbm.at[slab, pl.ds(ch * CH, CH)], dcur, scur).wait()
            for r in range(CH):
                j = ch * CH + r
                buf, sem = gbufs[r % 2], gsems[r % 2]
                nbuf, nsem = gbufs[(r + 1) % 2], gsems[(r + 1) % 2]
                if half == 1 and r == CH - 1:
                    @pl.when(g < npair - 1)
                    def _():
                        pltpu.async_copy(hw_hbm.at[src_v.at[j + 1]], nbuf, nsem)
                else:
                    pltpu.async_copy(hw_hbm.at[src_v.at[j + 1]], nbuf, nsem)
                pltpu.make_async_copy(hw_hbm.at[src_v.at[j]], buf, sem).wait()
                pltpu.sync_copy(buf, acc_sh.at[dcur.at[r]], add=True)

    plsc.subcore_barrier()
    pltpu.sync_copy(acc_sh.at[pl.ds(s * rpt, rpt)],
                    out_hbm.at[c, pl.ds(s * rpt, rpt)])


def _deg_kernel(n_pad, blk):
    rpt = n_pad // NS
    return pl.kernel(
        functools.partial(_deg_body, blk, rpt),
        out_type=jax.ShapeDtypeStruct((NC, n_pad), jnp.float32),
        mesh=_vector_mesh,
        scratch_types=[
            pltpu.VMEM((blk, K), jnp.int32),
            pltpu.VMEM((K,), jnp.float32),
            pltpu.VMEM((rpt,), jnp.float32),
            pltpu.VMEM_SHARED((n_pad,), jnp.float32),
        ],
    )


def _agg_kernel(n_pad, blk, d):
    rpt = n_pad // NS
    return pl.kernel(
        functools.partial(_agg_body, blk, rpt, d),
        out_type=jax.ShapeDtypeStruct((NC, n_pad, d), jnp.float32),
        mesh=_vector_mesh,
        scratch_types=[
            pltpu.VMEM((blk, K), jnp.int32),
            pltpu.VMEM((8, K), jnp.int32),
            pltpu.VMEM((8, K), jnp.int32),
            pltpu.VMEM((K, d), jnp.float32),
            pltpu.VMEM((K, d), jnp.float32),
            pltpu.VMEM((16, d), jnp.float32),
            pltpu.VMEM_SHARED((n_pad, d), jnp.float32),
            pltpu.SemaphoreType.DMA,
            pltpu.SemaphoreType.DMA,
            pltpu.SemaphoreType.DMA,
            pltpu.SemaphoreType.DMA,
        ],
    )


def _mm_body(x_ref, w_ref, o_ref):
    o_ref[...] = jnp.dot(x_ref[...], w_ref[...],
                         preferred_element_type=jnp.float32)


def _scale_body(degt_ref, xw_ref, o_ref):
    dinv = lax.rsqrt(degt_ref[...].sum(axis=1, keepdims=True) + 1.0)
    o_ref[...] = xw_ref[...] * dinv


def _mid_body(degt_ref, p0_ref, p1_ref, hws_ref, b_ref, w_ref, o_ref):
    dinv = lax.rsqrt(degt_ref[...].sum(axis=1, keepdims=True) + 1.0)
    h = (p0_ref[...] + p1_ref[...] + hws_ref[...]) * dinv + b_ref[...]
    h = jnp.maximum(h, 0.0)
    o_ref[...] = jnp.dot(h, w_ref[...],
                         preferred_element_type=jnp.float32) * dinv


def _final_body(degt_ref, q0_ref, q1_ref, hws_ref, b_ref, o_ref):
    dinv = lax.rsqrt(degt_ref[...].sum(axis=1, keepdims=True) + 1.0)
    o_ref[...] = (q0_ref[...] + q1_ref[...] + hws_ref[...]) * dinv + b_ref[...]


def kernel(x, edge_index, W1, b1, W2, b2):
    n, d = x.shape
    e = edge_index.shape[1]

    n_pad = ((n + NS * 16) // (NS * 16)) * (NS * 16)
    src = edge_index[0].astype(jnp.int32)
    dst = edge_index[1].astype(jnp.int32)

    blk = -(-e // (NW * K))
    blk = -(-blk // 16) * 16
    pad = NW * blk * K - e
    src3 = jnp.concatenate(
        [src, jnp.zeros((pad,), jnp.int32)]).reshape(NW, blk, K)
    dst3 = jnp.concatenate(
        [dst, jnp.full((pad,), n, jnp.int32)]).reshape(NW, blk, K)

    f32 = jnp.float32

    deg_parts = _deg_kernel(n_pad, blk)(dst3)
    xw1 = pl.pallas_call(
        _mm_body, out_shape=jax.ShapeDtypeStruct((n, d), f32))(x, W1)
    degt = deg_parts[:, :n].T

    hws1 = pl.pallas_call(
        _scale_body, out_shape=jax.ShapeDtypeStruct((n, d), f32))(degt, xw1)

    agg = _agg_kernel(n_pad, blk, d)
    p = agg(hws1, src3, dst3)
    hws2 = pl.pallas_call(
        _mid_body, out_shape=jax.ShapeDtypeStruct((n, d), f32))(
            degt, p[0, :n], p[1, :n], hws1, b1.reshape(1, d), W2)

    q = agg(hws2, src3, dst3)
    out = pl.pallas_call(
        _final_body, out_shape=jax.ShapeDtypeStruct((n, d), f32))(
            degt, q[0, :n], q[1, :n], hws2, b2.reshape(1, d))
    return out


# speedup vs baseline: 1.5079x; 1.5079x over previous
import functools

import jax
import jax.numpy as jnp
from jax import lax
from jax.experimental import pallas as pl
from jax.experimental.pallas import tpu as pltpu
from jax.experimental.pallas import tpu_sc as plsc

NC = 2
NS = 16
NW = NC * NS
K = 128
BLK0 = 104
BLK1 = 54

_vector_mesh = plsc.VectorSubcoreMesh(core_axis_name="c", subcore_axis_name="s")


def _deg_body(blk, rpt, dst_hbm, deg_out, idx_v, ones_v, z_v, deg_sh):
    c = lax.axis_index("c")
    s = lax.axis_index("s")
    slab = c * NS + s
    z16 = jnp.zeros((16,), jnp.float32)
    o16 = jnp.ones((16,), jnp.float32)
    for i in range(K // 16):
        ones_v[pl.ds(i * 16, 16)] = o16
    for i in range(rpt // 16):
        z_v[pl.ds(i * 16, 16)] = z16
    pltpu.sync_copy(z_v, deg_sh.at[pl.ds(s * rpt, rpt)])
    pltpu.sync_copy(dst_hbm.at[slab], idx_v)
    plsc.subcore_barrier()

    @pl.loop(0, blk)
    def _(j):
        pltpu.sync_copy(ones_v, deg_sh.at[idx_v.at[j]], add=True)

    plsc.subcore_barrier()
    pltpu.sync_copy(deg_sh.at[pl.ds(s * rpt, rpt)],
                    deg_out.at[c, pl.ds(s * rpt, rpt)])


def _agg_body(rpt, d, hw_hbm, src_hbm, dst_hbm, out_hbm,
              src_v, dst_v, buf, z_v, acc_sh):
    c = lax.axis_index("c")
    s = lax.axis_index("s")
    z16 = jnp.zeros((16,), jnp.float32)
    for r in range(16):
        for i in range(d // 16):
            z_v[r, pl.ds(i * 16, 16)] = z16

    @pl.loop(0, rpt // 16)
    def _(i):
        pltpu.sync_copy(z_v, acc_sh.at[pl.ds(s * rpt + i * 16, 16)])

    base = c * BLK0
    cnt = jnp.where(c == 0, BLK0, BLK1)
    pltpu.sync_copy(src_hbm.at[s, pl.ds(base, BLK0)], src_v)
    pltpu.sync_copy(dst_hbm.at[s, pl.ds(base, BLK0)], dst_v)
    plsc.subcore_barrier()

    @pl.loop(0, cnt)
    def _(j):
        pltpu.sync_copy(hw_hbm.at[src_v.at[j]], buf)
        pltpu.sync_copy(buf, acc_sh.at[dst_v.at[j]], add=True)

    plsc.subcore_barrier()
    pltpu.sync_copy(acc_sh.at[pl.ds(s * rpt, rpt)],
                    out_hbm.at[c, pl.ds(s * rpt, rpt)])


def _deg_kernel(n_pad, blk):
    rpt = n_pad // NS
    return pl.kernel(
        functools.partial(_deg_body, blk, rpt),
        out_type=jax.ShapeDtypeStruct((NC, n_pad), jnp.float32),
        mesh=_vector_mesh,
        scratch_types=[
            pltpu.VMEM((blk, K), jnp.int32),
            pltpu.VMEM((K,), jnp.float32),
            pltpu.VMEM((rpt,), jnp.float32),
            pltpu.VMEM_SHARED((n_pad,), jnp.float32),
        ],
    )


def _agg_kernel(n_pad, d):
    rpt = n_pad // NS
    return pl.kernel(
        functools.partial(_agg_body, rpt, d),
        out_type=jax.ShapeDtypeStruct((NC, n_pad, d), jnp.float32),
        mesh=_vector_mesh,
        scratch_types=[
            pltpu.VMEM((BLK0, K), jnp.int32),
            pltpu.VMEM((BLK0, K), jnp.int32),
            pltpu.VMEM((K, d), jnp.float32),
            pltpu.VMEM((16, d), jnp.float32),
            pltpu.VMEM_SHARED((n_pad, d), jnp.float32),
        ],
    )


def _mm_body(x_ref, w_ref, o_ref):
    o_ref[...] = jnp.dot(x_ref[...], w_ref[...],
                         preferred_element_type=jnp.float32)


def _scale_body(degt_ref, xw_ref, o_ref):
    dinv = lax.rsqrt(degt_ref[...].sum(axis=1, keepdims=True) + 1.0)
    o_ref[...] = xw_ref[...] * dinv


def _mid_body(degt_ref, p0_ref, p1_ref, hws_ref, b_ref, w_ref, o_ref):
    dinv = lax.rsqrt(degt_ref[...].sum(axis=1, keepdims=True) + 1.0)
    h = (p0_ref[...] + p1_ref[...] + hws_ref[...]) * dinv + b_ref[...]
    h = jnp.maximum(h, 0.0)
    o_ref[...] = jnp.dot(h, w_ref[...],
                         preferred_element_type=jnp.float32) * dinv


def _final_body(degt_ref, q0_ref, q1_ref, hws_ref, b_ref, o_ref):
    dinv = lax.rsqrt(degt_ref[...].sum(axis=1, keepdims=True) + 1.0)
    o_ref[...] = (q0_ref[...] + q1_ref[...] + hws_ref[...]) * dinv + b_ref[...]


def _asym_slabs(v, e0, pad_val):
    r0 = v[:e0].reshape(NS, BLK0, K)
    r1 = v[e0:].reshape(NS, BLK1, K)
    r1 = jnp.pad(r1, ((0, 0), (0, BLK0 - BLK1), (0, 0)),
                 constant_values=pad_val)
    return jnp.concatenate([r0, r1], axis=1)


def kernel(x, edge_index, W1, b1, W2, b2):
    n, d = x.shape
    e = edge_index.shape[1]

    n_pad = ((n + NS * 16) // (NS * 16)) * (NS * 16)
    src = edge_index[0].astype(jnp.int32)
    dst = edge_index[1].astype(jnp.int32)

    blk_d = -(-e // (NW * K))
    pad_d = NW * blk_d * K - e
    dst3_d = jnp.concatenate(
        [dst, jnp.full((pad_d,), n, jnp.int32)]).reshape(NW, blk_d, K)

    e_pad = NS * (BLK0 + BLK1) * K
    e0 = NS * BLK0 * K
    srcp = jnp.concatenate([src, jnp.zeros((e_pad - e,), jnp.int32)])
    dstp = jnp.concatenate([dst, jnp.full((e_pad - e,), n, jnp.int32)])
    src3 = _asym_slabs(srcp, e0, 0)
    dst3 = _asym_slabs(dstp, e0, n)

    f32 = jnp.float32
    x_pad = jnp.pad(x, ((0, n_pad - n), (0, 0)))

    deg_parts = _deg_kernel(n_pad, blk_d)(dst3_d)
    xw1 = pl.pallas_call(
        _mm_body, out_shape=jax.ShapeDtypeStruct((n_pad, d), f32))(x_pad, W1)
    degt = deg_parts.T

    hws1 = pl.pallas_call(
        _scale_body, out_shape=jax.ShapeDtypeStruct((n_pad, d), f32))(degt, xw1)

    agg = _agg_kernel(n_pad, d)
    p = agg(hws1, src3, dst3)
    hws2 = pl.pallas_call(
        _mid_body, out_shape=jax.ShapeDtypeStruct((n_pad, d), f32))(
            degt, p[0], p[1], hws1, b1.reshape(1, d), W2)

    q = agg(hws2, src3, dst3)
    out = pl.pallas_call(
        _final_body, out_shape=jax.ShapeDtypeStruct((n_pad, d), f32))(
            degt, q[0], q[1], hws2, b2.reshape(1, d))
    return out[:n]
